# baseline (speedup 1.0000x reference)
import functools

import jax
import jax.numpy as jnp
from jax import lax
from jax.experimental import pallas as pl
from jax.experimental.pallas import tpu as pltpu
from jax.experimental.pallas import tpu_sc as plsc

D_MODEL = 768
N_TIERS = 8
B = 8192

NC = 2
NS = 16
NW = NC * NS
B_PER_W = B // NW
CHUNK = 32
N_CHUNKS = B_PER_W // CHUNK
N_SLICES = D_MODEL // 16

TB = 1024


def _router_body(q_ref, w_ref, logits_ref, gw_ref, idx0_ref, idx1_ref, w0x_ref):
    q = q_ref[...]
    w = w_ref[...]
    logits = lax.dot_general(
        q, w, (((1,), (1,)), ((), ())), preferred_element_type=jnp.float32
    )
    logits_ref[...] = logits

    iota = lax.broadcasted_iota(jnp.int32, (TB, N_TIERS), 1)
    m1 = jnp.max(logits, axis=1, keepdims=True)
    i1 = jnp.min(jnp.where(logits == m1, iota, N_TIERS), axis=1, keepdims=True)
    masked = jnp.where(iota == i1, jnp.float32(-jnp.inf), logits)
    m2 = jnp.max(masked, axis=1, keepdims=True)
    i2 = jnp.min(jnp.where(masked == m2, iota, N_TIERS), axis=1, keepdims=True)

    e = jnp.exp(m2 - m1)
    w0 = 1.0 / (1.0 + e)
    w1 = e / (1.0 + e)

    gw_ref[...] = jnp.where(iota == i1, w0, 0.0) + jnp.where(iota == i2, w1, 0.0)

    gid = pl.program_id(0) * TB + lax.broadcasted_iota(jnp.int32, (TB, 1), 0)
    idx0_ref[...] = i1 * B + gid
    idx1_ref[...] = i2 * B + gid
    w0x_ref[...] = jnp.broadcast_to(w0, (TB, 16))


_router = pl.pallas_call(
    _router_body,
    grid=(B // TB,),
    in_specs=[
        pl.BlockSpec((TB, D_MODEL), lambda i: (i, 0)),
        pl.BlockSpec((N_TIERS, D_MODEL), lambda i: (0, 0)),
    ],
    out_specs=[
        pl.BlockSpec((TB, N_TIERS), lambda i: (i, 0)),
        pl.BlockSpec((TB, N_TIERS), lambda i: (i, 0)),
        pl.BlockSpec((TB, 1), lambda i: (i, 0)),
        pl.BlockSpec((TB, 1), lambda i: (i, 0)),
        pl.BlockSpec((TB, 16), lambda i: (i, 0)),
    ],
    out_shape=[
        jax.ShapeDtypeStruct((B, N_TIERS), jnp.float32),
        jax.ShapeDtypeStruct((B, N_TIERS), jnp.float32),
        jax.ShapeDtypeStruct((B, 1), jnp.int32),
        jax.ShapeDtypeStruct((B, 1), jnp.int32),
        jax.ShapeDtypeStruct((B, 16), jnp.float32),
    ],
)


def _combine_body(table, idx0, idx1, w0x, out,
                  idx0_v, idx1_v, w0_v, rows0, rows1, sem0, sem1):
    wid = lax.axis_index("s") * NC + lax.axis_index("c")
    base_w = wid * B_PER_W
    pltpu.sync_copy(idx0.at[pl.ds(base_w, B_PER_W)], idx0_v)
    pltpu.sync_copy(idx1.at[pl.ds(base_w, B_PER_W)], idx1_v)
    pltpu.sync_copy(w0x.at[pl.ds(base_w * 16, B_PER_W * 16)], w0_v)

    sems = (sem0, sem1)

    def start_gathers(c):
        p = c & 1
        isl = pl.ds(c * CHUNK, CHUNK)
        cpa = pltpu.async_copy(table.at[idx0_v.at[isl]], rows0.at[p], sems[p])
        cpb = pltpu.async_copy(table.at[idx1_v.at[isl]], rows1.at[p], sems[p])
        return cpa, cpb

    pending = start_gathers(0)
    for c in range(N_CHUNKS):
        p = c & 1
        cpa, cpb = pending
        if c + 1 < N_CHUNKS:
            nxt = start_gathers(c + 1)
        cpa.wait()
        cpb.wait()

        def tok_body(t, carry):
            w = w0_v[pl.ds((c * CHUNK + t) * 16, 16)]
            for d in range(N_SLICES):
                sl = pl.ds(d * 16, 16)
                a = rows0[p, t, sl]
                b = rows1[p, t, sl]
                rows0[p, t, sl] = b + w * (a - b)
            return carry

        lax.fori_loop(0, CHUNK, tok_body, 0)
        pltpu.sync_copy(rows0.at[p], out.at[pl.ds(base_w + c * CHUNK, CHUNK)])
        if c + 1 < N_CHUNKS:
            pending = nxt


@functools.lru_cache(maxsize=1)
def _make_combine():
    return pl.kernel(
        _combine_body,
        out_type=jax.ShapeDtypeStruct((B, D_MODEL), jnp.float32),
        mesh=plsc.VectorSubcoreMesh(core_axis_name="c", subcore_axis_name="s"),
        scratch_types=[
            pltpu.VMEM((B_PER_W,), jnp.int32),
            pltpu.VMEM((B_PER_W,), jnp.int32),
            pltpu.VMEM((B_PER_W * 16,), jnp.float32),
            pltpu.VMEM((2, CHUNK, D_MODEL), jnp.float32),
            pltpu.VMEM((2, CHUNK, D_MODEL), jnp.float32),
            pltpu.SemaphoreType.DMA,
            pltpu.SemaphoreType.DMA,
        ],
    )


def kernel(tier_outputs, query, W):
    logits, gate_weights, idx0, idx1, w0x = _router(query, W)
    table = tier_outputs.reshape(N_TIERS * B, D_MODEL)
    merged = _make_combine()(table, idx0.reshape(B), idx1.reshape(B),
                             w0x.reshape(B * 16))
    return merged, gate_weights, logits


# speedup vs baseline: 1.4238x; 1.4238x over previous
import functools

import jax
import jax.numpy as jnp
from jax import lax
from jax.experimental import pallas as pl
from jax.experimental.pallas import tpu as pltpu
from jax.experimental.pallas import tpu_sc as plsc

D_MODEL = 768
N_TIERS = 8
B = 8192

NC = 2
NS = 16
NW = NC * NS
B_PER_W = B // NW
CHUNK = 32
N_CHUNKS = B_PER_W // CHUNK
N_SLICES = D_MODEL // 16

TB = 1024


def _router_body(q_ref, w_ref, logits_ref, gw_ref, idx0_ref, idx1_ref, w0x_ref):
    q = q_ref[...]
    w = w_ref[...]
    logits = lax.dot_general(
        q, w, (((1,), (1,)), ((), ())), preferred_element_type=jnp.float32
    )
    logits_ref[...] = logits

    iota = lax.broadcasted_iota(jnp.int32, (TB, N_TIERS), 1)
    m1 = jnp.max(logits, axis=1, keepdims=True)
    i1 = jnp.min(jnp.where(logits == m1, iota, N_TIERS), axis=1, keepdims=True)
    masked = jnp.where(iota == i1, jnp.float32(-jnp.inf), logits)
    m2 = jnp.max(masked, axis=1, keepdims=True)
    i2 = jnp.min(jnp.where(masked == m2, iota, N_TIERS), axis=1, keepdims=True)

    e = jnp.exp(m2 - m1)
    w0 = 1.0 / (1.0 + e)
    w1 = e / (1.0 + e)

    gw_ref[...] = jnp.where(iota == i1, w0, 0.0) + jnp.where(iota == i2, w1, 0.0)

    gid = pl.program_id(0) * TB + lax.broadcasted_iota(jnp.int32, (TB, 1), 0)
    idx0_ref[...] = i1 * B + gid
    idx1_ref[...] = i2 * B + gid
    w0x_ref[...] = jnp.broadcast_to(w0, (TB, 16))


_router = pl.pallas_call(
    _router_body,
    grid=(B // TB,),
    in_specs=[
        pl.BlockSpec((TB, D_MODEL), lambda i: (i, 0)),
        pl.BlockSpec((N_TIERS, D_MODEL), lambda i: (0, 0)),
    ],
    out_specs=[
        pl.BlockSpec((TB, N_TIERS), lambda i: (i, 0)),
        pl.BlockSpec((TB, N_TIERS), lambda i: (i, 0)),
        pl.BlockSpec((TB, 1), lambda i: (i, 0)),
        pl.BlockSpec((TB, 1), lambda i: (i, 0)),
        pl.BlockSpec((TB, 16), lambda i: (i, 0)),
    ],
    out_shape=[
        jax.ShapeDtypeStruct((B, N_TIERS), jnp.float32),
        jax.ShapeDtypeStruct((B, N_TIERS), jnp.float32),
        jax.ShapeDtypeStruct((B, 1), jnp.int32),
        jax.ShapeDtypeStruct((B, 1), jnp.int32),
        jax.ShapeDtypeStruct((B, 16), jnp.float32),
    ],
)


def _combine_body(table, idx0, idx1, w0x, out,
                  idx0_a, idx0_b, idx1_a, idx1_b, w0_v,
                  rows0_a, rows0_b, rows1_a, rows1_b, sem0, sem1):
    wid = lax.axis_index("s") * NC + lax.axis_index("c")
    base_w = wid * B_PER_W
    pltpu.sync_copy(w0x.at[pl.ds(base_w * 16, B_PER_W * 16)], w0_v)

    idx0_p = (idx0_a, idx0_b)
    idx1_p = (idx1_a, idx1_b)
    rows0_p = (rows0_a, rows0_b)
    rows1_p = (rows1_a, rows1_b)
    sems = (sem0, sem1)

    def start_gathers(c):
        p = c & 1
        base = base_w + c * CHUNK
        pltpu.sync_copy(idx0.at[pl.ds(base, CHUNK)], idx0_p[p])
        pltpu.sync_copy(idx1.at[pl.ds(base, CHUNK)], idx1_p[p])
        cpa = pltpu.async_copy(table.at[idx0_p[p]], rows0_p[p], sems[p])
        cpb = pltpu.async_copy(table.at[idx1_p[p]], rows1_p[p], sems[p])
        return cpa, cpb

    pending = start_gathers(0)
    for c in range(N_CHUNKS):
        p = c & 1
        cpa, cpb = pending
        if c + 1 < N_CHUNKS:
            nxt = start_gathers(c + 1)
        cpa.wait()
        cpb.wait()
        rows0 = rows0_p[p]
        rows1 = rows1_p[p]

        def tok_body(t, carry):
            w = w0_v[pl.ds((c * CHUNK + t) * 16, 16)]
            for d in range(N_SLICES):
                sl = pl.ds(d * 16, 16)
                a = rows0[t, sl]
                b = rows1[t, sl]
                rows0[t, sl] = b + w * (a - b)
            return carry

        lax.fori_loop(0, CHUNK, tok_body, 0)
        pltpu.sync_copy(rows0, out.at[pl.ds(base_w + c * CHUNK, CHUNK)])
        if c + 1 < N_CHUNKS:
            pending = nxt


@functools.lru_cache(maxsize=1)
def _make_combine():
    return pl.kernel(
        _combine_body,
        out_type=jax.ShapeDtypeStruct((B, D_MODEL), jnp.float32),
        mesh=plsc.VectorSubcoreMesh(core_axis_name="c", subcore_axis_name="s"),
        scratch_types=[
            pltpu.VMEM((CHUNK,), jnp.int32),
            pltpu.VMEM((CHUNK,), jnp.int32),
            pltpu.VMEM((CHUNK,), jnp.int32),
            pltpu.VMEM((CHUNK,), jnp.int32),
            pltpu.VMEM((B_PER_W * 16,), jnp.float32),
            pltpu.VMEM((CHUNK, D_MODEL), jnp.float32),
            pltpu.VMEM((CHUNK, D_MODEL), jnp.float32),
            pltpu.VMEM((CHUNK, D_MODEL), jnp.float32),
            pltpu.VMEM((CHUNK, D_MODEL), jnp.float32),
            pltpu.SemaphoreType.DMA,
            pltpu.SemaphoreType.DMA,
        ],
    )


def kernel(tier_outputs, query, W):
    logits, gate_weights, idx0, idx1, w0x = _router(query, W)
    table = tier_outputs.reshape(N_TIERS * B, D_MODEL)
    merged = _make_combine()(table, idx0.reshape(B), idx1.reshape(B),
                             w0x.reshape(B * 16))
    return merged, gate_weights, logits


# speedup vs baseline: 1.6443x; 1.1549x over previous
import functools

import jax
import jax.numpy as jnp
from jax import lax
from jax.experimental import pallas as pl
from jax.experimental.pallas import tpu as pltpu
from jax.experimental.pallas import tpu_sc as plsc

D_MODEL = 768
N_TIERS = 8
B = 8192

NC = 2
NS = 16
NW = NC * NS
B_PER_W = B // NW
CHUNK = 32
N_CHUNKS = B_PER_W // CHUNK
N_SLICES = D_MODEL // 16

TB = 1024


def _router_body(q_ref, w_ref, logits_ref, gw_ref, idx0_ref, idx1_ref, w0_ref):
    q = q_ref[...]
    w = w_ref[...]
    lt = lax.dot_general(
        w, q, (((1,), (1,)), ((), ())), preferred_element_type=jnp.float32
    )

    iota = lax.broadcasted_iota(jnp.int32, (N_TIERS, TB), 0)
    m1 = jnp.max(lt, axis=0, keepdims=True)
    i1 = jnp.min(jnp.where(lt == m1, iota, N_TIERS), axis=0, keepdims=True)
    masked = jnp.where(iota == i1, jnp.float32(-jnp.inf), lt)
    m2 = jnp.max(masked, axis=0, keepdims=True)
    i2 = jnp.min(jnp.where(masked == m2, iota, N_TIERS), axis=0, keepdims=True)

    e = jnp.exp(m2 - m1)
    w0 = 1.0 / (1.0 + e)
    w1 = e / (1.0 + e)

    gw_t = jnp.where(iota == i1, w0, 0.0) + jnp.where(iota == i2, w1, 0.0)
    logits_ref[...] = lt.T
    gw_ref[...] = gw_t.T

    gid = pl.program_id(0) * TB + lax.broadcasted_iota(jnp.int32, (1, TB), 1)
    idx0_ref[...] = (i1 * B + gid).reshape(TB)
    idx1_ref[...] = (i2 * B + gid).reshape(TB)
    w0_ref[...] = w0.reshape(TB)


_router = pl.pallas_call(
    _router_body,
    grid=(B // TB,),
    in_specs=[
        pl.BlockSpec((TB, D_MODEL), lambda i: (i, 0)),
        pl.BlockSpec((N_TIERS, D_MODEL), lambda i: (0, 0)),
    ],
    out_specs=[
        pl.BlockSpec((TB, N_TIERS), lambda i: (i, 0)),
        pl.BlockSpec((TB, N_TIERS), lambda i: (i, 0)),
        pl.BlockSpec((TB,), lambda i: (i,)),
        pl.BlockSpec((TB,), lambda i: (i,)),
        pl.BlockSpec((TB,), lambda i: (i,)),
    ],
    out_shape=[
        jax.ShapeDtypeStruct((B, N_TIERS), jnp.float32),
        jax.ShapeDtypeStruct((B, N_TIERS), jnp.float32),
        jax.ShapeDtypeStruct((B,), jnp.int32),
        jax.ShapeDtypeStruct((B,), jnp.int32),
        jax.ShapeDtypeStruct((B,), jnp.float32),
    ],
)


def _combine_body(table, idx0, idx1, w0x, out,
                  idx0_a, idx0_b, idx1_a, idx1_b, w0_v,
                  rows0_a, rows0_b, rows1_a, rows1_b, sem0, sem1):
    wid = lax.axis_index("s") * NC + lax.axis_index("c")
    base_w = wid * B_PER_W
    pltpu.sync_copy(w0x.at[pl.ds(base_w * 16, B_PER_W * 16)], w0_v)

    idx0_p = (idx0_a, idx0_b)
    idx1_p = (idx1_a, idx1_b)
    rows0_p = (rows0_a, rows0_b)
    rows1_p = (rows1_a, rows1_b)
    sems = (sem0, sem1)

    def start_gathers(c):
        p = c & 1
        base = base_w + c * CHUNK
        pltpu.sync_copy(idx0.at[pl.ds(base, CHUNK)], idx0_p[p])
        pltpu.sync_copy(idx1.at[pl.ds(base, CHUNK)], idx1_p[p])
        cpa = pltpu.async_copy(table.at[idx0_p[p]], rows0_p[p], sems[p])
        cpb = pltpu.async_copy(table.at[idx1_p[p]], rows1_p[p], sems[p])
        return cpa, cpb

    pending = start_gathers(0)
    for c in range(N_CHUNKS):
        p = c & 1
        cpa, cpb = pending
        if c + 1 < N_CHUNKS:
            nxt = start_gathers(c + 1)
        cpa.wait()
        cpb.wait()
        rows0 = rows0_p[p]
        rows1 = rows1_p[p]

        def tok_body(t, carry):
            w = w0_v[pl.ds((c * CHUNK + t) * 16, 16)]
            for d in range(N_SLICES):
                sl = pl.ds(d * 16, 16)
                a = rows0[t, sl]
                b = rows1[t, sl]
                rows0[t, sl] = b + w * (a - b)
            return carry

        lax.fori_loop(0, CHUNK, tok_body, 0)
        pltpu.sync_copy(rows0, out.at[pl.ds(base_w + c * CHUNK, CHUNK)])
        if c + 1 < N_CHUNKS:
            pending = nxt


@functools.lru_cache(maxsize=1)
def _make_combine():
    return pl.kernel(
        _combine_body,
        out_type=jax.ShapeDtypeStruct((B, D_MODEL), jnp.float32),
        mesh=plsc.VectorSubcoreMesh(core_axis_name="c", subcore_axis_name="s"),
        scratch_types=[
            pltpu.VMEM((CHUNK,), jnp.int32),
            pltpu.VMEM((CHUNK,), jnp.int32),
            pltpu.VMEM((CHUNK,), jnp.int32),
            pltpu.VMEM((CHUNK,), jnp.int32),
            pltpu.VMEM((B_PER_W * 16,), jnp.float32),
            pltpu.VMEM((CHUNK, D_MODEL), jnp.float32),
            pltpu.VMEM((CHUNK, D_MODEL), jnp.float32),
            pltpu.VMEM((CHUNK, D_MODEL), jnp.float32),
            pltpu.VMEM((CHUNK, D_MODEL), jnp.float32),
            pltpu.SemaphoreType.DMA,
            pltpu.SemaphoreType.DMA,
        ],
    )


def kernel(tier_outputs, query, W):
    logits, gate_weights, idx0, idx1, w0 = _router(query, W)
    table = tier_outputs.reshape(N_TIERS * B, D_MODEL)
    w0x = jnp.repeat(w0, 16)
    merged = _make_combine()(table, idx0, idx1, w0x)
    return merged, gate_weights, logits
